# baseline (speedup 1.0000x reference)
import functools

import jax
import jax.numpy as jnp
from jax import lax
from jax.experimental import pallas as pl
from jax.experimental.pallas import tpu as pltpu
from jax.experimental.pallas import tpu_sc as plsc

_NN = {'p': 20000, 's': 20000, 'g': 100000}
_ETYPES = {
    'pg': ('p', 'g', 128000), 'ps': ('p', 's', 64000), 'sp': ('s', 'p', 64000),
    'sg': ('s', 'g', 128000), 'gp': ('g', 'p', 64000), 'gs': ('g', 's', 64000),
    'gg': ('g', 'g', 160000)}
_TKEYS = list(_ETYPES)

_B = 128
_DUMP = 64
_ZB = 40
_NTILE = 16
_CH = 4


def _part_count(n_dst, d):
    p = 2
    while (n_dst // p) * d * 4 > 6_400_000:
        p *= 2
    return p


def _build_passes(d, with_cnt):
    e_off = {}
    off = 0
    for k in _TKEYS:
        e_off[k] = off
        off += _ETYPES[k][2]
    e_tot = off
    out_off = {}
    off = 0
    for k in _TKEYS:
        out_off[k] = off
        off += _NN[_ETYPES[k][1]]
    sum_rows = off

    passes = []
    for k in _TKEYS:
        _, dst_t, e = _ETYPES[k]
        n_dst = _NN[dst_t]
        p_cnt = _part_count(n_dst, d)
        for j in range(p_cnt // 2):
            passes.append([e_off[k], e // _B, j, p_cnt // 2,
                           n_dst // p_cnt, out_off[k], True])
    if with_cnt:
        for k in _TKEYS:
            _, dst_t, e = _ETYPES[k]
            n_dst = _NN[dst_t]
            p_cnt = _part_count(n_dst, d)
            for j in range(p_cnt // 2):
                passes.append([e_off[k], e // _B, j, p_cnt // 2,
                               n_dst // p_cnt, sum_rows + out_off[k], False])
    part_max = max(p[4] for p in passes)
    out_rows = sum_rows * (2 if with_cnt else 1)
    return passes, part_max, out_rows, e_tot, out_off, sum_rows


def _segsum_body(passes, d, with_cnt, *refs):
    if with_cnt:
        (xcat, rows, cols, zref, oref, out_ref,
         table, rowflat, colflat, idxbuf, rowsb, xrow, xcol, zbuf, onesb,
         gsem, xsem) = refs
    else:
        (xcat, rows, cols, zref, out_ref,
         table, rowflat, colflat, idxbuf, rowsb, xrow, xcol, zbuf, onesb,
         gsem, xsem) = refs

    core = lax.axis_index("c")
    sub = lax.axis_index("s")

    pltpu.sync_copy(zref, zbuf)
    if with_cnt:
        pltpu.sync_copy(oref, onesb)

    def munge(cref, b_off, base, part_rows):
        for i in range(_B // 16):
            c = cref[pl.ds(b_off + i * 16, 16)]
            inpart = (c >= base) & (c < base + part_rows)
            idx = jnp.where(inpart, c - base,
                            part_rows + (c & (_DUMP - 1)))
            idxbuf[0, pl.ds(i * 16, 16)] = idx

    def scat(ridref, base, part_rows, cref, coff):
        munge(cref, coff, base, part_rows)
        pltpu.async_copy(xcat.at[ridref], rowsb.at[0], xsem).wait()
        pltpu.sync_copy(rowsb.at[0], table.at[idxbuf.at[0]], add=True)

    def pass_body(e_base, nblk, jpart, npasses, part_rows, out_base, gather):
        base = (core * npasses + jpart) * part_rows
        q = nblk // _NTILE
        r = nblk - q * _NTILE

        nzb = (part_rows + _DUMP) // _ZB + 2
        nzi = nzb // _NTILE + jnp.where(sub < nzb - (nzb // _NTILE) * _NTILE,
                                        1, 0)

        def zbody(i, c2):
            z = sub + i * _NTILE
            pltpu.sync_copy(zbuf.at[pl.ds(0, _ZB)],
                            table.at[pl.ds(z * _ZB, _ZB)])
            return c2
        lax.fori_loop(0, nzi, zbody, 0)
        plsc.subcore_barrier()

        start_blk = sub * q + jnp.minimum(sub, r)
        start_e = e_base + start_blk * _B

        nfull = q // _CH

        def chunk_body(ch, c2):
            b0e = start_e + ch * (_CH * _B)
            pltpu.sync_copy(cols.at[pl.ds(b0e, _CH * _B)], colflat)
            if gather:
                pltpu.sync_copy(rows.at[pl.ds(b0e, _CH * _B)], rowflat)
                pltpu.async_copy(xcat.at[rowflat.at[pl.ds(0, _B)]],
                                 rowsb.at[0], gsem.at[0])
                for i in range(_CH):
                    sl = i % 2
                    if i + 1 < _CH:
                        pltpu.async_copy(
                            xcat.at[rowflat.at[pl.ds((i + 1) * _B, _B)]],
                            rowsb.at[1 - sl], gsem.at[1 - sl])
                    munge(colflat, i * _B, base, part_rows)
                    pltpu.make_async_copy(
                        xcat.at[rowflat.at[pl.ds(i * _B, _B)]],
                        rowsb.at[sl], gsem.at[sl]).wait()
                    pltpu.sync_copy(rowsb.at[sl], table.at[idxbuf.at[0]],
                                    add=True)
            else:
                for i in range(_CH):
                    munge(colflat, i * _B, base, part_rows)
                    pltpu.sync_copy(onesb, table.at[idxbuf.at[0]],
                                    add=True)
            return c2
        lax.fori_loop(0, nfull, chunk_body, 0)

        def tail_body(tb, c2):
            tbe = start_e + (nfull * _CH + tb) * _B
            pltpu.sync_copy(cols.at[pl.ds(tbe, _B)], xcol)
            if gather:
                pltpu.sync_copy(rows.at[pl.ds(tbe, _B)], xrow)
                scat(xrow, base, part_rows, xcol, 0)
            else:
                munge(xcol, 0, base, part_rows)
                pltpu.sync_copy(onesb, table.at[idxbuf.at[0]], add=True)
            return c2
        lax.fori_loop(0, q - nfull * _CH, tail_body, 0)

        @pl.when(sub < r)
        def _():
            xbe = e_base + (start_blk + q) * _B
            pltpu.sync_copy(cols.at[pl.ds(xbe, _B)], xcol)
            if gather:
                pltpu.sync_copy(rows.at[pl.ds(xbe, _B)], xrow)
                scat(xrow, base, part_rows, xcol, 0)
            else:
                munge(xcol, 0, base, part_rows)
                pltpu.sync_copy(onesb, table.at[idxbuf.at[0]], add=True)

        plsc.subcore_barrier()

        nwb = part_rows // _ZB
        nwi = nwb // _NTILE + jnp.where(sub < nwb - (nwb // _NTILE) * _NTILE,
                                        1, 0)

        def wbody(i, c2):
            z = sub + i * _NTILE
            pltpu.sync_copy(
                table.at[pl.ds(z * _ZB, _ZB)],
                out_ref.at[pl.ds(out_base + base + z * _ZB, _ZB)])
            return c2
        lax.fori_loop(0, nwi, wbody, 0)
        plsc.subcore_barrier()

    for p in passes:
        pass_body(*p)


def _make_segsum(d, with_cnt):
    passes, part_max, out_rows, e_tot, _, _ = _build_passes(d, with_cnt)
    out_type = jax.ShapeDtypeStruct((out_rows, d), jnp.float32)

    scratch = [
        pltpu.VMEM_SHARED((part_max + _DUMP + 2 * _ZB, d), jnp.float32),
        pltpu.VMEM((_CH * _B,), jnp.int32),
        pltpu.VMEM((_CH * _B,), jnp.int32),
        pltpu.VMEM((1, _B), jnp.int32),
        pltpu.VMEM((2, _B, d), jnp.float32),
        pltpu.VMEM((_B,), jnp.int32),
        pltpu.VMEM((_B,), jnp.int32),
        pltpu.VMEM((_ZB, d), jnp.float32),
        pltpu.VMEM((_B, d) if with_cnt else (8, d), jnp.float32),
        pltpu.SemaphoreType.DMA((2,)),
        pltpu.SemaphoreType.DMA,
    ]
    body = functools.partial(_segsum_body, passes, d, with_cnt)
    return pl.kernel(
        body, out_type=out_type,
        mesh=plsc.VectorSubcoreMesh(core_axis_name="c", subcore_axis_name="s"),
        scratch_types=scratch,
        compiler_params=pltpu.CompilerParams(use_tc_tiling_on_sc=False))


def _segsums(xd, eis, d, with_cnt):
    passes, _, _, e_tot, out_off, sum_rows = _build_passes(d, with_cnt)
    fn = _make_segsum(d, with_cnt)

    n_p, n_s = _NN['p'], _NN['s']
    src_off = {'p': 0, 's': n_p, 'g': n_p + n_s}
    xcat = jnp.concatenate([xd['p'], xd['s'], xd['g']], axis=0)

    rows_cat = jnp.concatenate(
        [eis[k][0] + src_off[_ETYPES[k][0]] for k in _TKEYS])
    cols_cat = jnp.concatenate([eis[k][1] for k in _TKEYS])

    zeros = jnp.zeros((_ZB, d), jnp.float32)
    args = [xcat, rows_cat, cols_cat, zeros]
    if with_cnt:
        args.append(jnp.ones((_B, d), jnp.float32))

    big = fn(*args)
    cnts = big[sum_rows:, :1] if with_cnt else None
    return big, cnts


_BLK = 2000


def _row_spec(w, off=0):
    ob = off // _BLK
    return pl.BlockSpec((_BLK, w), lambda i, ob=ob: (ob + i, 0))


def _full_spec(shape):
    nd = len(shape)
    return pl.BlockSpec(shape, lambda i: (0,) * nd)


def _acc_stats(i, st_ref, y):
    ps = jnp.sum(y, 0, keepdims=True)
    pq = jnp.sum(y * y, 0, keepdims=True)
    st = jnp.concatenate([ps, pq, jnp.zeros((6, y.shape[1]), y.dtype)], 0)

    @pl.when(i == 0)
    def _():
        st_ref[...] = jnp.zeros_like(st_ref)
    st_ref[...] += st


def _mm_stats_kernel(x_ref, w_ref, b_ref, o_ref, st_ref):
    i = pl.program_id(0)
    y = jnp.dot(x_ref[...], w_ref[...],
                preferred_element_type=jnp.float32) + b_ref[...]
    o_ref[...] = y
    _acc_stats(i, st_ref, y)


def _mm_stats(x, w, b):
    n, din = x.shape
    dout = w.shape[1]
    return pl.pallas_call(
        _mm_stats_kernel,
        grid=(n // _BLK,),
        in_specs=[_row_spec(din), _full_spec((din, dout)),
                  _full_spec((1, dout))],
        out_specs=[_row_spec(dout), _full_spec((8, dout))],
        out_shape=[jax.ShapeDtypeStruct((n, dout), jnp.float32),
                   jax.ShapeDtypeStruct((8, dout), jnp.float32)],
    )(x, w, b.reshape(1, -1))


def _mm_statsonly_kernel(x_ref, w_ref, b_ref, st_ref):
    i = pl.program_id(0)
    y = jnp.dot(x_ref[...], w_ref[...],
                preferred_element_type=jnp.float32) + b_ref[...]
    _acc_stats(i, st_ref, y)


def _mm_statsonly(x, w, b):
    n, din = x.shape
    dout = w.shape[1]
    return pl.pallas_call(
        _mm_statsonly_kernel,
        grid=(n // _BLK,),
        in_specs=[_row_spec(din), _full_spec((din, dout)),
                  _full_spec((1, dout))],
        out_specs=_full_spec((8, dout)),
        out_shape=jax.ShapeDtypeStruct((8, dout), jnp.float32),
    )(x, w, b.reshape(1, -1))


def _affine(st, n, g, b, eps=1e-5, double=False):
    mu = st[0] / n
    var = st[1] / n - mu * mu
    sc = g / jnp.sqrt(var + eps)
    sh = b - mu * sc
    if double:
        sc, sh = 2 * sc, 2 * sh
    return sc, sh


def _affine_relu_kernel(y_ref, c_ref, o_ref):
    o_ref[...] = jnp.maximum(y_ref[...] * c_ref[0:1] + c_ref[1:2], 0.0)


def _affine_relu(y, sc, sh):
    n, dout = y.shape
    consts = jnp.stack([sc, sh])
    return pl.pallas_call(
        _affine_relu_kernel,
        grid=(n // _BLK,),
        in_specs=[_row_spec(dout), _full_spec((2, dout))],
        out_specs=_row_spec(dout),
        out_shape=jax.ShapeDtypeStruct((n, dout), jnp.float32),
    )(y, consts)


def _ea_apply_kernel(x_ref, w_ref, b_ref, c_ref, o_ref):
    y = jnp.dot(x_ref[...], w_ref[...],
                preferred_element_type=jnp.float32) + b_ref[...]
    o_ref[...] = jnp.maximum(y * c_ref[0:1] + c_ref[1:2], 0.0)


def _ea_apply(x, w, b, sc, sh):
    n, din = x.shape
    dout = w.shape[1]
    consts = jnp.stack([sc, sh])
    return pl.pallas_call(
        _ea_apply_kernel,
        grid=(n // _BLK,),
        in_specs=[_row_spec(din), _full_spec((din, dout)),
                  _full_spec((1, dout)), _full_spec((2, dout))],
        out_specs=_row_spec(dout),
        out_shape=jax.ShapeDtypeStruct((n, dout), jnp.float32),
    )(x, w, b.reshape(1, -1), consts)


def _phase1_kernel(sum_ref, cnt_ref, x_ref, ea_ref, m_ref, wl_ref, wr_ref,
                   bl_ref, we_ref, be_ref, wt_ref, bt_ref, wa_ref, ba_ref,
                   wg_ref, bg_ref, o_ref, st_ref):
    i = pl.program_id(0)
    agg = sum_ref[...] / cnt_ref[...]
    out = (jnp.dot(agg, wl_ref[...], preferred_element_type=jnp.float32)
           + bl_ref[...]
           + jnp.dot(x_ref[...], wr_ref[...],
                     preferred_element_type=jnp.float32))
    ea = ea_ref[...]
    emb = jnp.maximum(
        jnp.dot(ea, we_ref[...], preferred_element_type=jnp.float32)
        + be_ref[...], 0.0)
    t_emb = jnp.dot(emb, wt_ref[...],
                    preferred_element_type=jnp.float32) + bt_ref[...]
    t_attr = jnp.dot(ea, wa_ref[...],
                     preferred_element_type=jnp.float32) + ba_ref[...]
    t = t_emb + t_attr
    wg = wg_ref[...]
    o = t.shape[1]
    g0 = (jnp.dot(t, wg[o:2 * o], preferred_element_type=jnp.float32)
          + jnp.dot(t_attr, wg[2 * o:], preferred_element_type=jnp.float32)
          + bg_ref[...])
    gate = jax.nn.sigmoid(
        jnp.dot(out, wg[:o], preferred_element_type=jnp.float32) + g0)
    out2 = out + m_ref[...] * (gate * t)
    o_ref[...] = out2
    _acc_stats(i, st_ref, out2)


def _phase1(big, cnt_cat, off, x_dst, ea_w, mask, p, d):
    n = x_dst.shape[0]
    o = p['sage']['wl'].shape[1]
    return pl.pallas_call(
        _phase1_kernel,
        grid=(n // _BLK,),
        in_specs=[
            _row_spec(d, off), _row_spec(1, off), _row_spec(d),
            _row_spec(32), _row_spec(1),
            _full_spec((d, o)), _full_spec((d, o)), _full_spec((1, o)),
            _full_spec((32, 16)), _full_spec((1, 16)),
            _full_spec((16, o)), _full_spec((1, o)),
            _full_spec((32, o)), _full_spec((1, o)),
            _full_spec((3 * o, o)), _full_spec((1, o)),
        ],
        out_specs=[_row_spec(o), _full_spec((8, o))],
        out_shape=[jax.ShapeDtypeStruct((n, o), jnp.float32),
                   jax.ShapeDtypeStruct((8, o), jnp.float32)],
    )(big, cnt_cat, x_dst, ea_w, mask,
      p['sage']['wl'], p['sage']['wr'], p['sage']['bl'].reshape(1, -1),
      p['emb']['w'], p['emb']['b'].reshape(1, -1),
      p['temb']['w'], p['temb']['b'].reshape(1, -1),
      p['tattr']['w'], p['tattr']['b'].reshape(1, -1),
      p['gate']['w'], p['gate']['b'].reshape(1, -1))


def _phase2_3_kernel(a_ref, b2_ref, c2_ref, x_ref, w_ref, bl_ref, cs_ref,
                     o_ref):
    acc = (jnp.dot(x_ref[...], w_ref[...],
                   preferred_element_type=jnp.float32) + bl_ref[...])
    cs = cs_ref[...]
    for j, r in enumerate((a_ref, b2_ref, c2_ref)):
        acc += jnp.maximum(r[...] * cs[2 * j:2 * j + 1]
                           + cs[2 * j + 1:2 * j + 2], 0.0)
    o_ref[...] = jnp.maximum(acc, 0.0)


def _phase2_2_kernel(a_ref, b2_ref, x_ref, w_ref, bl_ref, cs_ref, o_ref):
    acc = (jnp.dot(x_ref[...], w_ref[...],
                   preferred_element_type=jnp.float32) + bl_ref[...])
    cs = cs_ref[...]
    for j, r in enumerate((a_ref, b2_ref)):
        acc += jnp.maximum(r[...] * cs[2 * j:2 * j + 1]
                           + cs[2 * j + 1:2 * j + 2], 0.0)
    o_ref[...] = jnp.maximum(acc, 0.0)


def _phase2(contribs, consts, x_self, w_self, bl_self, o):
    n = x_self.shape[0]
    d = x_self.shape[1]
    cs = jnp.concatenate([jnp.stack([sc, sh]) for sc, sh in consts])
    kfn = _phase2_3_kernel if len(contribs) == 3 else _phase2_2_kernel
    return pl.pallas_call(
        kfn,
        grid=(n // _BLK,),
        in_specs=[_row_spec(o) for _ in contribs]
        + [_row_spec(d), _full_spec((d, o)), _full_spec((1, o)),
           _full_spec((2 * len(contribs), o))],
        out_specs=_row_spec(o),
        out_shape=jax.ShapeDtypeStruct((n, o), jnp.float32),
    )(*contribs, x_self, w_self, bl_self.reshape(1, -1), cs)


def _hetero_pallas(xd, big, cnt_cat, meta, pl_, d, out_off):
    outs2 = {}
    stats = {}
    for k, (s, t_dst, _) in _ETYPES.items():
        win_mask, ea_w = meta[k]
        o2, st = _phase1(big, cnt_cat, out_off[k], xd[t_dst], ea_w,
                         win_mask, pl_[k], d)
        outs2[k] = o2
        stats[k] = st
    res = {}
    for t in ['p', 's', 'g']:
        ks = [k for k, v in _ETYPES.items() if v[1] == t]
        consts = []
        for k in ks:
            n_k = _NN[t]
            bn = pl_[k]['bn']
            consts.append(_affine(stats[k], n_k, bn['g'], bn['b'],
                                  double=True))
        sp = pl_['self_' + t]
        res[t] = _phase2([outs2[k] for k in ks], consts, xd[t],
                         sp['wl'] + sp['wr'], sp['bl'], 64)
    return res


def _final_lin_kernel(x_ref, w_ref, b_ref, a_ref, o_ref):
    y = x_ref[...] @ w_ref[...] + b_ref[0, 0]
    a = a_ref[0, 0]
    o_ref[...] = jnp.where(y >= 0, y, a * y)


def _final_lin(x, w, b, a):
    n = x.shape[0]
    blk = 2000
    return pl.pallas_call(
        _final_lin_kernel,
        grid=(n // blk,),
        in_specs=[
            pl.BlockSpec((blk, x.shape[1]), lambda i: (i, 0)),
            pl.BlockSpec((x.shape[1], 1), lambda i: (0, 0)),
            pl.BlockSpec((1, 1), lambda i: (0, 0)),
            pl.BlockSpec((1, 1), lambda i: (0, 0)),
        ],
        out_specs=pl.BlockSpec((blk, 1), lambda i: (i, 0)),
        out_shape=jax.ShapeDtypeStruct((n, 1), jnp.float32),
    )(x, w, b.reshape(1, 1), a.reshape(1, 1))


def kernel(x_pfas, x_sw, x_gw, eas, params, eis):
    xin = {'p': x_pfas, 's': x_sw, 'g': x_gw}
    nn = {t: v.shape[0] for t, v in xin.items()}

    xd = {}
    for t, x in xin.items():
        pr, pb = params['node_red'][t], params['node_bn'][t]
        y, st = _mm_stats(x, pr['w'], pr['b'])
        sc, sh = _affine(st, nn[t], pb['g'], pb['b'])
        xd[t] = _affine_relu(y, sc, sh)

    meta = {}
    for k, (s, d, e) in _ETYPES.items():
        col = eis[k][1]
        n = nn[d]
        win = jax.ops.segment_max(jnp.arange(e, dtype=jnp.int32), col,
                                  num_segments=n)
        mask = (win >= 0) & (win < e)
        winc = jnp.where(mask, win, 0)
        pr, pb = params['edge_red'][k], params['edge_bn'][k]
        st = _mm_statsonly(eas[k], pr['w'], pr['b'])
        sc, sh = _affine(st, e, pb['g'], pb['b'])
        ea_w = _ea_apply(eas[k][winc], pr['w'], pr['b'], sc, sh)
        meta[k] = (mask.astype(jnp.float32).reshape(-1, 1), ea_w)

    _, _, _, _, out_off, sum_rows = _build_passes(32, True)

    big1, cnts1 = _segsums(xd, eis, 32, True)
    cnt_cat = jnp.maximum(cnts1, 1.0)
    xd = _hetero_pallas(xd, big1, cnt_cat, meta, params['conv1'], 32,
                        out_off)

    big2, _ = _segsums(xd, eis, 64, False)
    xd = _hetero_pallas(xd, big2, cnt_cat, meta, params['conv2'], 64,
                        out_off)

    w, b, a = params['lin']['w'], params['lin']['b'], params['prelu']
    gw = _final_lin(xd['g'], w, b, a)
    sw = _final_lin(xd['s'], w, b, a)
    return gw, sw, xd['p']


# speedup vs baseline: 1.5186x; 1.5186x over previous
import functools

import jax
import jax.numpy as jnp
from jax import lax
from jax.experimental import pallas as pl
from jax.experimental.pallas import tpu as pltpu
from jax.experimental.pallas import tpu_sc as plsc

_NN = {'p': 20000, 's': 20000, 'g': 100000}
_ETYPES = {
    'pg': ('p', 'g', 128000), 'ps': ('p', 's', 64000), 'sp': ('s', 'p', 64000),
    'sg': ('s', 'g', 128000), 'gp': ('g', 'p', 64000), 'gs': ('g', 's', 64000),
    'gg': ('g', 'g', 160000)}
_TKEYS = list(_ETYPES)

_B = 128
_DUMP = 64
_ZB = 40
_NTILE = 16
_CH = 4


def _part_count(n_dst, d):
    p = 2
    while (n_dst // p) * d * 4 > 6_400_000:
        p *= 2
    return p


def _build_passes(d, with_cnt):
    e_off = {}
    off = 0
    for k in _TKEYS:
        e_off[k] = off
        off += _ETYPES[k][2]
    e_tot = off
    out_off = {}
    off = 0
    for k in _TKEYS:
        out_off[k] = off
        off += _NN[_ETYPES[k][1]]
    sum_rows = off

    passes = []
    for k in _TKEYS:
        _, dst_t, e = _ETYPES[k]
        n_dst = _NN[dst_t]
        p_cnt = _part_count(n_dst, d)
        for j in range(p_cnt // 2):
            passes.append([e_off[k], e // _B, j, p_cnt // 2,
                           n_dst // p_cnt, out_off[k], True])
    if with_cnt:
        for k in _TKEYS:
            _, dst_t, e = _ETYPES[k]
            n_dst = _NN[dst_t]
            p_cnt = _part_count(n_dst, d)
            for j in range(p_cnt // 2):
                passes.append([e_off[k], e // _B, j, p_cnt // 2,
                               n_dst // p_cnt, sum_rows + out_off[k], False])
    part_max = max(p[4] for p in passes)
    out_rows = sum_rows * (2 if with_cnt else 1)
    return passes, part_max, out_rows, e_tot, out_off, sum_rows


def _segsum_body(passes, d, with_cnt, *refs):
    if with_cnt:
        (xcat, rows, cols, zref, oref, out_ref,
         table, rowflat, colflat, idxbuf, rowsb, xrow, xcol, zbuf, onesb,
         gsem, xsem) = refs
    else:
        (xcat, rows, cols, zref, out_ref,
         table, rowflat, colflat, idxbuf, rowsb, xrow, xcol, zbuf, onesb,
         gsem, xsem) = refs

    core = lax.axis_index("c")
    sub = lax.axis_index("s")

    pltpu.sync_copy(zref, zbuf)
    if with_cnt:
        pltpu.sync_copy(oref, onesb)

    def munge(cref, b_off, base, part_rows):
        for i in range(_B // 16):
            c = cref[pl.ds(b_off + i * 16, 16)]
            inpart = (c >= base) & (c < base + part_rows)
            idx = jnp.where(inpart, c - base,
                            part_rows + (c & (_DUMP - 1)))
            idxbuf[0, pl.ds(i * 16, 16)] = idx

    def scat(ridref, base, part_rows, cref, coff):
        munge(cref, coff, base, part_rows)
        pltpu.async_copy(xcat.at[ridref], rowsb.at[0], xsem).wait()
        pltpu.sync_copy(rowsb.at[0], table.at[idxbuf.at[0]], add=True)

    def pass_body(e_base, nblk, jpart, npasses, part_rows, out_base, gather):
        base = (core * npasses + jpart) * part_rows
        q = nblk // _NTILE
        r = nblk - q * _NTILE

        nzb = (part_rows + _DUMP) // _ZB + 2
        nzi = nzb // _NTILE + jnp.where(sub < nzb - (nzb // _NTILE) * _NTILE,
                                        1, 0)

        def zbody(i, c2):
            z = sub + i * _NTILE
            pltpu.sync_copy(zbuf.at[pl.ds(0, _ZB)],
                            table.at[pl.ds(z * _ZB, _ZB)])
            return c2
        lax.fori_loop(0, nzi, zbody, 0)
        plsc.subcore_barrier()

        start_blk = sub * q + jnp.minimum(sub, r)
        start_e = e_base + start_blk * _B

        nfull = q // _CH

        def chunk_body(ch, c2):
            b0e = start_e + ch * (_CH * _B)
            pltpu.sync_copy(cols.at[pl.ds(b0e, _CH * _B)], colflat)
            if gather:
                pltpu.sync_copy(rows.at[pl.ds(b0e, _CH * _B)], rowflat)
                pltpu.async_copy(xcat.at[rowflat.at[pl.ds(0, _B)]],
                                 rowsb.at[0], gsem.at[0])
                for i in range(_CH):
                    sl = i % 2
                    if i + 1 < _CH:
                        pltpu.async_copy(
                            xcat.at[rowflat.at[pl.ds((i + 1) * _B, _B)]],
                            rowsb.at[1 - sl], gsem.at[1 - sl])
                    munge(colflat, i * _B, base, part_rows)
                    pltpu.make_async_copy(
                        xcat.at[rowflat.at[pl.ds(i * _B, _B)]],
                        rowsb.at[sl], gsem.at[sl]).wait()
                    pltpu.sync_copy(rowsb.at[sl], table.at[idxbuf.at[0]],
                                    add=True)
            else:
                for i in range(_CH):
                    munge(colflat, i * _B, base, part_rows)
                    pltpu.sync_copy(onesb, table.at[idxbuf.at[0]],
                                    add=True)
            return c2
        lax.fori_loop(0, nfull, chunk_body, 0)

        def tail_body(tb, c2):
            tbe = start_e + (nfull * _CH + tb) * _B
            pltpu.sync_copy(cols.at[pl.ds(tbe, _B)], xcol)
            if gather:
                pltpu.sync_copy(rows.at[pl.ds(tbe, _B)], xrow)
                scat(xrow, base, part_rows, xcol, 0)
            else:
                munge(xcol, 0, base, part_rows)
                pltpu.sync_copy(onesb, table.at[idxbuf.at[0]], add=True)
            return c2
        lax.fori_loop(0, q - nfull * _CH, tail_body, 0)

        @pl.when(sub < r)
        def _():
            xbe = e_base + (start_blk + q) * _B
            pltpu.sync_copy(cols.at[pl.ds(xbe, _B)], xcol)
            if gather:
                pltpu.sync_copy(rows.at[pl.ds(xbe, _B)], xrow)
                scat(xrow, base, part_rows, xcol, 0)
            else:
                munge(xcol, 0, base, part_rows)
                pltpu.sync_copy(onesb, table.at[idxbuf.at[0]], add=True)

        plsc.subcore_barrier()

        nwb = part_rows // _ZB
        nwi = nwb // _NTILE + jnp.where(sub < nwb - (nwb // _NTILE) * _NTILE,
                                        1, 0)

        def wbody(i, c2):
            z = sub + i * _NTILE
            pltpu.sync_copy(
                table.at[pl.ds(z * _ZB, _ZB)],
                out_ref.at[pl.ds(out_base + base + z * _ZB, _ZB)])
            return c2
        lax.fori_loop(0, nwi, wbody, 0)
        plsc.subcore_barrier()

    for p in passes:
        pass_body(*p)


def _make_segsum(d, with_cnt):
    passes, part_max, out_rows, e_tot, _, _ = _build_passes(d, with_cnt)
    out_type = jax.ShapeDtypeStruct((out_rows, d), jnp.float32)

    scratch = [
        pltpu.VMEM_SHARED((part_max + _DUMP + 2 * _ZB, d), jnp.float32),
        pltpu.VMEM((_CH * _B,), jnp.int32),
        pltpu.VMEM((_CH * _B,), jnp.int32),
        pltpu.VMEM((1, _B), jnp.int32),
        pltpu.VMEM((2, _B, d), jnp.float32),
        pltpu.VMEM((_B,), jnp.int32),
        pltpu.VMEM((_B,), jnp.int32),
        pltpu.VMEM((_ZB, d), jnp.float32),
        pltpu.VMEM((_B, d) if with_cnt else (8, d), jnp.float32),
        pltpu.SemaphoreType.DMA((2,)),
        pltpu.SemaphoreType.DMA,
    ]
    body = functools.partial(_segsum_body, passes, d, with_cnt)
    return pl.kernel(
        body, out_type=out_type,
        mesh=plsc.VectorSubcoreMesh(core_axis_name="c", subcore_axis_name="s"),
        scratch_types=scratch,
        compiler_params=pltpu.CompilerParams(use_tc_tiling_on_sc=False))


def _segsums(xd, eis, d, with_cnt):
    passes, _, _, e_tot, out_off, sum_rows = _build_passes(d, with_cnt)
    fn = _make_segsum(d, with_cnt)

    n_p, n_s = _NN['p'], _NN['s']
    src_off = {'p': 0, 's': n_p, 'g': n_p + n_s}
    xcat = jnp.concatenate([xd['p'], xd['s'], xd['g']], axis=0)

    rows_cat = jnp.concatenate(
        [eis[k][0] + src_off[_ETYPES[k][0]] for k in _TKEYS])
    cols_cat = jnp.concatenate([eis[k][1] for k in _TKEYS])

    zeros = jnp.zeros((_ZB, d), jnp.float32)
    args = [xcat, rows_cat, cols_cat, zeros]
    if with_cnt:
        args.append(jnp.ones((_B, d), jnp.float32))

    big = fn(*args)
    cnts = big[sum_rows:, :1] if with_cnt else None
    return big, cnts


def _gather_body(nblks, *refs):
    eas_r = refs[:7]
    win_r = refs[7:14]
    out_r = refs[14:21]
    ibuf, rowsb, gsem = refs[21:]

    core = lax.axis_index("c")
    sub = lax.axis_index("s")
    wid = sub * 2 + core

    for t in range(7):
        ea_ref, win_ref, o_ref = eas_r[t], win_r[t], out_r[t]
        nblk = nblks[t]
        q, r = nblk // 32, nblk % 32
        start = wid * q + jnp.minimum(wid, r)
        nmine = q + jnp.where(wid < r, 1, 0)
        nch = (nmine + _CH - 1) // _CH

        def chunk_body(ch, c2, ea_ref=ea_ref, win_ref=win_ref, o_ref=o_ref,
                       start=start, nmine=nmine):
            b0 = ch * _CH
            pltpu.sync_copy(
                win_ref.at[pl.ds((start + b0) * _B, _CH * _B)], ibuf)
            pltpu.async_copy(ea_ref.at[ibuf.at[pl.ds(0, _B)]],
                             rowsb.at[0], gsem.at[0])
            for i in range(_CH):
                sl = i % 2
                if i + 1 < _CH:
                    @pl.when(b0 + i + 1 < nmine)
                    def _():
                        pltpu.async_copy(
                            ea_ref.at[ibuf.at[pl.ds((i + 1) * _B, _B)]],
                            rowsb.at[1 - sl], gsem.at[1 - sl])

                @pl.when(b0 + i < nmine)
                def _():
                    pltpu.make_async_copy(
                        ea_ref.at[ibuf.at[pl.ds(i * _B, _B)]],
                        rowsb.at[sl], gsem.at[sl]).wait()
                    pltpu.sync_copy(
                        rowsb.at[sl],
                        o_ref.at[pl.ds((start + b0 + i) * _B, _B)])
            return c2
        lax.fori_loop(0, nch, chunk_body, 0)


def _sc_gather(ead_list, win_list):
    npads = [w.shape[0] for w in win_list]
    nblks = [n // _B for n in npads]
    out_type = [jax.ShapeDtypeStruct((n, 16), jnp.float32) for n in npads]
    scratch = [
        pltpu.VMEM((_CH * _B,), jnp.int32),
        pltpu.VMEM((2, _B, 16), jnp.float32),
        pltpu.SemaphoreType.DMA((2,)),
    ]
    fn = pl.kernel(
        functools.partial(_gather_body, nblks),
        out_type=out_type,
        mesh=plsc.VectorSubcoreMesh(core_axis_name="c", subcore_axis_name="s"),
        scratch_types=scratch,
        compiler_params=pltpu.CompilerParams(use_tc_tiling_on_sc=False))
    return fn(*ead_list, *win_list)


_BLK = 2000


def _row_spec(w, off=0):
    ob = off // _BLK
    return pl.BlockSpec((_BLK, w), lambda i, ob=ob: (ob + i, 0))


def _full_spec(shape):
    nd = len(shape)
    return pl.BlockSpec(shape, lambda i: (0,) * nd)


def _acc_stats(i, st_ref, y):
    ps = jnp.sum(y, 0, keepdims=True)
    pq = jnp.sum(y * y, 0, keepdims=True)
    st = jnp.concatenate([ps, pq, jnp.zeros((6, y.shape[1]), y.dtype)], 0)

    @pl.when(i == 0)
    def _():
        st_ref[...] = jnp.zeros_like(st_ref)
    st_ref[...] += st


def _mm_stats_kernel(x_ref, w_ref, b_ref, o_ref, st_ref):
    i = pl.program_id(0)
    y = jnp.dot(x_ref[...], w_ref[...],
                preferred_element_type=jnp.float32) + b_ref[...]
    o_ref[...] = y
    _acc_stats(i, st_ref, y)


def _mm_stats(x, w, b):
    n, din = x.shape
    dout = w.shape[1]
    return pl.pallas_call(
        _mm_stats_kernel,
        grid=(n // _BLK,),
        in_specs=[_row_spec(din), _full_spec((din, dout)),
                  _full_spec((1, dout))],
        out_specs=[_row_spec(dout), _full_spec((8, dout))],
        out_shape=[jax.ShapeDtypeStruct((n, dout), jnp.float32),
                   jax.ShapeDtypeStruct((8, dout), jnp.float32)],
    )(x, w, b.reshape(1, -1))


def _mm_statsonly_kernel(x_ref, w_ref, b_ref, st_ref):
    i = pl.program_id(0)
    y = jnp.dot(x_ref[...], w_ref[...],
                preferred_element_type=jnp.float32) + b_ref[...]
    _acc_stats(i, st_ref, y)


def _mm_statsonly(x, w, b):
    n, din = x.shape
    dout = w.shape[1]
    return pl.pallas_call(
        _mm_statsonly_kernel,
        grid=(n // _BLK,),
        in_specs=[_row_spec(din), _full_spec((din, dout)),
                  _full_spec((1, dout))],
        out_specs=_full_spec((8, dout)),
        out_shape=jax.ShapeDtypeStruct((8, dout), jnp.float32),
    )(x, w, b.reshape(1, -1))


def _affine(st, n, g, b, eps=1e-5, double=False):
    mu = st[0] / n
    var = st[1] / n - mu * mu
    sc = g / jnp.sqrt(var + eps)
    sh = b - mu * sc
    if double:
        sc, sh = 2 * sc, 2 * sh
    return sc, sh


def _affine_relu_kernel(y_ref, c_ref, o_ref):
    o_ref[...] = jnp.maximum(y_ref[...] * c_ref[0:1] + c_ref[1:2], 0.0)


def _affine_relu(y, sc, sh):
    n, dout = y.shape
    consts = jnp.stack([sc, sh])
    return pl.pallas_call(
        _affine_relu_kernel,
        grid=(n // _BLK,),
        in_specs=[_row_spec(dout), _full_spec((2, dout))],
        out_specs=_row_spec(dout),
        out_shape=jax.ShapeDtypeStruct((n, dout), jnp.float32),
    )(y, consts)


def _ea_apply_kernel(x_ref, w_ref, b_ref, c_ref, o_ref):
    y = jnp.dot(x_ref[...], w_ref[...],
                preferred_element_type=jnp.float32) + b_ref[...]
    o_ref[...] = jnp.maximum(y * c_ref[0:1] + c_ref[1:2], 0.0)


def _ea_apply(x, w, b, sc, sh):
    n, din = x.shape
    dout = w.shape[1]
    consts = jnp.stack([sc, sh])
    return pl.pallas_call(
        _ea_apply_kernel,
        grid=(n // _BLK,),
        in_specs=[_row_spec(din), _full_spec((din, dout)),
                  _full_spec((1, dout)), _full_spec((2, dout))],
        out_specs=_row_spec(dout),
        out_shape=jax.ShapeDtypeStruct((n, dout), jnp.float32),
    )(x, w, b.reshape(1, -1), consts)


def _phase1_kernel(sum_ref, cnt_ref, x_ref, ea_ref, m_ref, wl_ref, wr_ref,
                   bl_ref, we_ref, be_ref, wt_ref, bt_ref, wa_ref, ba_ref,
                   wg_ref, bg_ref, o_ref, st_ref):
    i = pl.program_id(0)
    agg = sum_ref[...] / cnt_ref[...]
    out = (jnp.dot(agg, wl_ref[...], preferred_element_type=jnp.float32)
           + bl_ref[...]
           + jnp.dot(x_ref[...], wr_ref[...],
                     preferred_element_type=jnp.float32))
    ea = ea_ref[...]
    emb = jnp.maximum(
        jnp.dot(ea, we_ref[...], preferred_element_type=jnp.float32)
        + be_ref[...], 0.0)
    t_emb = jnp.dot(emb, wt_ref[...],
                    preferred_element_type=jnp.float32) + bt_ref[...]
    t_attr = jnp.dot(ea, wa_ref[...],
                     preferred_element_type=jnp.float32) + ba_ref[...]
    t = t_emb + t_attr
    wg = wg_ref[...]
    o = t.shape[1]
    g0 = (jnp.dot(t, wg[o:2 * o], preferred_element_type=jnp.float32)
          + jnp.dot(t_attr, wg[2 * o:], preferred_element_type=jnp.float32)
          + bg_ref[...])
    gate = jax.nn.sigmoid(
        jnp.dot(out, wg[:o], preferred_element_type=jnp.float32) + g0)
    out2 = out + m_ref[...] * (gate * t)
    o_ref[...] = out2
    _acc_stats(i, st_ref, out2)


def _phase1(big, cnt_cat, off, x_dst, ea_w, mask, p, d):
    n = x_dst.shape[0]
    o = p['sage']['wl'].shape[1]
    return pl.pallas_call(
        _phase1_kernel,
        grid=(n // _BLK,),
        in_specs=[
            _row_spec(d, off), _row_spec(1, off), _row_spec(d),
            _row_spec(32), _row_spec(1),
            _full_spec((d, o)), _full_spec((d, o)), _full_spec((1, o)),
            _full_spec((32, 16)), _full_spec((1, 16)),
            _full_spec((16, o)), _full_spec((1, o)),
            _full_spec((32, o)), _full_spec((1, o)),
            _full_spec((3 * o, o)), _full_spec((1, o)),
        ],
        out_specs=[_row_spec(o), _full_spec((8, o))],
        out_shape=[jax.ShapeDtypeStruct((n, o), jnp.float32),
                   jax.ShapeDtypeStruct((8, o), jnp.float32)],
    )(big, cnt_cat, x_dst, ea_w, mask,
      p['sage']['wl'], p['sage']['wr'], p['sage']['bl'].reshape(1, -1),
      p['emb']['w'], p['emb']['b'].reshape(1, -1),
      p['temb']['w'], p['temb']['b'].reshape(1, -1),
      p['tattr']['w'], p['tattr']['b'].reshape(1, -1),
      p['gate']['w'], p['gate']['b'].reshape(1, -1))


def _phase2_3_kernel(a_ref, b2_ref, c2_ref, x_ref, w_ref, bl_ref, cs_ref,
                     o_ref):
    acc = (jnp.dot(x_ref[...], w_ref[...],
                   preferred_element_type=jnp.float32) + bl_ref[...])
    cs = cs_ref[...]
    for j, r in enumerate((a_ref, b2_ref, c2_ref)):
        acc += jnp.maximum(r[...] * cs[2 * j:2 * j + 1]
                           + cs[2 * j + 1:2 * j + 2], 0.0)
    o_ref[...] = jnp.maximum(acc, 0.0)


def _phase2_2_kernel(a_ref, b2_ref, x_ref, w_ref, bl_ref, cs_ref, o_ref):
    acc = (jnp.dot(x_ref[...], w_ref[...],
                   preferred_element_type=jnp.float32) + bl_ref[...])
    cs = cs_ref[...]
    for j, r in enumerate((a_ref, b2_ref)):
        acc += jnp.maximum(r[...] * cs[2 * j:2 * j + 1]
                           + cs[2 * j + 1:2 * j + 2], 0.0)
    o_ref[...] = jnp.maximum(acc, 0.0)


def _phase2(contribs, consts, x_self, w_self, bl_self, o):
    n = x_self.shape[0]
    d = x_self.shape[1]
    cs = jnp.concatenate([jnp.stack([sc, sh]) for sc, sh in consts])
    kfn = _phase2_3_kernel if len(contribs) == 3 else _phase2_2_kernel
    return pl.pallas_call(
        kfn,
        grid=(n // _BLK,),
        in_specs=[_row_spec(o) for _ in contribs]
        + [_row_spec(d), _full_spec((d, o)), _full_spec((1, o)),
           _full_spec((2 * len(contribs), o))],
        out_specs=_row_spec(o),
        out_shape=jax.ShapeDtypeStruct((n, o), jnp.float32),
    )(*contribs, x_self, w_self, bl_self.reshape(1, -1), cs)


def _hetero_pallas(xd, big, cnt_cat, meta, pl_, d, out_off):
    outs2 = {}
    stats = {}
    for k, (s, t_dst, _) in _ETYPES.items():
        win_mask, ea_w = meta[k]
        o2, st = _phase1(big, cnt_cat, out_off[k], xd[t_dst], ea_w,
                         win_mask, pl_[k], d)
        outs2[k] = o2
        stats[k] = st
    res = {}
    for t in ['p', 's', 'g']:
        ks = [k for k, v in _ETYPES.items() if v[1] == t]
        consts = []
        for k in ks:
            n_k = _NN[t]
            bn = pl_[k]['bn']
            consts.append(_affine(stats[k], n_k, bn['g'], bn['b'],
                                  double=True))
        sp = pl_['self_' + t]
        res[t] = _phase2([outs2[k] for k in ks], consts, xd[t],
                         sp['wl'] + sp['wr'], sp['bl'], 64)
    return res


def _final_lin_kernel(x_ref, w_ref, b_ref, a_ref, o_ref):
    y = x_ref[...] @ w_ref[...] + b_ref[0, 0]
    a = a_ref[0, 0]
    o_ref[...] = jnp.where(y >= 0, y, a * y)


def _final_lin(x, w, b, a):
    n = x.shape[0]
    blk = 2000
    return pl.pallas_call(
        _final_lin_kernel,
        grid=(n // blk,),
        in_specs=[
            pl.BlockSpec((blk, x.shape[1]), lambda i: (i, 0)),
            pl.BlockSpec((x.shape[1], 1), lambda i: (0, 0)),
            pl.BlockSpec((1, 1), lambda i: (0, 0)),
            pl.BlockSpec((1, 1), lambda i: (0, 0)),
        ],
        out_specs=pl.BlockSpec((blk, 1), lambda i: (i, 0)),
        out_shape=jax.ShapeDtypeStruct((n, 1), jnp.float32),
    )(x, w, b.reshape(1, 1), a.reshape(1, 1))


def kernel(x_pfas, x_sw, x_gw, eas, params, eis):
    xin = {'p': x_pfas, 's': x_sw, 'g': x_gw}
    nn = {t: v.shape[0] for t, v in xin.items()}

    xd = {}
    for t, x in xin.items():
        pr, pb = params['node_red'][t], params['node_bn'][t]
        y, st = _mm_stats(x, pr['w'], pr['b'])
        sc, sh = _affine(st, nn[t], pb['g'], pb['b'])
        xd[t] = _affine_relu(y, sc, sh)

    masks, wincs = {}, []
    gpad = _CH * _B
    for k, (s, d, e) in _ETYPES.items():
        col = eis[k][1]
        n = nn[d]
        win = jax.ops.segment_max(jnp.arange(e, dtype=jnp.int32), col,
                                  num_segments=n)
        mask = (win >= 0) & (win < e)
        winc = jnp.where(mask, win, 0)
        masks[k] = mask.astype(jnp.float32).reshape(-1, 1)
        npad = ((n + gpad - 1) // gpad + 1) * gpad
        wincs.append(jnp.pad(winc, (0, npad - n)))

    raw = _sc_gather([eas[k] for k in _TKEYS], wincs)
    meta = {}
    for i, (k, (s, d, e)) in enumerate(_ETYPES.items()):
        pr, pb = params['edge_red'][k], params['edge_bn'][k]
        st = _mm_statsonly(eas[k], pr['w'], pr['b'])
        sc, sh = _affine(st, e, pb['g'], pb['b'])
        ea_w = _ea_apply(raw[i][:nn[d]], pr['w'], pr['b'], sc, sh)
        meta[k] = (masks[k], ea_w)

    _, _, _, _, out_off, sum_rows = _build_passes(32, True)

    big1, cnts1 = _segsums(xd, eis, 32, True)
    cnt_cat = jnp.maximum(cnts1, 1.0)
    xd = _hetero_pallas(xd, big1, cnt_cat, meta, params['conv1'], 32,
                        out_off)

    big2, _ = _segsums(xd, eis, 64, False)
    xd = _hetero_pallas(xd, big2, cnt_cat, meta, params['conv2'], 64,
                        out_off)

    w, b, a = params['lin']['w'], params['lin']['b'], params['prelu']
    gw = _final_lin(xd['g'], w, b, a)
    sw = _final_lin(xd['s'], w, b, a)
    return gw, sw, xd['p']


# speedup vs baseline: 1.9047x; 1.2542x over previous
import functools

import jax
import jax.numpy as jnp
from jax import lax
from jax.experimental import pallas as pl
from jax.experimental.pallas import tpu as pltpu
from jax.experimental.pallas import tpu_sc as plsc

_NN = {'p': 20000, 's': 20000, 'g': 100000}
_ETYPES = {
    'pg': ('p', 'g', 128000), 'ps': ('p', 's', 64000), 'sp': ('s', 'p', 64000),
    'sg': ('s', 'g', 128000), 'gp': ('g', 'p', 64000), 'gs': ('g', 's', 64000),
    'gg': ('g', 'g', 160000)}
_TKEYS = list(_ETYPES)

_B = 128
_DUMP = 64
_ZB = 40
_NTILE = 16
_CH = 4


def _part_count(n_dst, d):
    p = 2
    while (n_dst // p) * d * 4 > 6_400_000:
        p *= 2
    return p


def _build_passes(d, with_cnt):
    e_off = {}
    off = 0
    for k in _TKEYS:
        e_off[k] = off
        off += _ETYPES[k][2]
    e_tot = off
    out_off = {}
    off = 0
    for k in _TKEYS:
        out_off[k] = off
        off += _NN[_ETYPES[k][1]]
    sum_rows = off

    passes = []
    for k in _TKEYS:
        _, dst_t, e = _ETYPES[k]
        n_dst = _NN[dst_t]
        p_cnt = _part_count(n_dst, d)
        for j in range(p_cnt // 2):
            passes.append([e_off[k], e // _B, j, p_cnt // 2,
                           n_dst // p_cnt, out_off[k], True])
    if with_cnt:
        for k in _TKEYS:
            _, dst_t, e = _ETYPES[k]
            n_dst = _NN[dst_t]
            p_cnt = _part_count(n_dst, d)
            for j in range(p_cnt // 2):
                passes.append([e_off[k], e // _B, j, p_cnt // 2,
                               n_dst // p_cnt, sum_rows + out_off[k], False])
    part_max = max(p[4] for p in passes)
    out_rows = sum_rows * (2 if with_cnt else 1)
    return passes, part_max, out_rows, e_tot, out_off, sum_rows


def _segsum_body(passes, d, with_cnt, *refs):
    if with_cnt:
        (xcat, rows, cols, zref, oref, out_ref,
         table, rowflat, colflat, idxbuf, rowsb, xrow, xcol, zbuf, onesb,
         gsem, xsem) = refs
    else:
        (xcat, rows, cols, zref, out_ref,
         table, rowflat, colflat, idxbuf, rowsb, xrow, xcol, zbuf, onesb,
         gsem, xsem) = refs

    core = lax.axis_index("c")
    sub = lax.axis_index("s")

    pltpu.sync_copy(zref, zbuf)
    if with_cnt:
        pltpu.sync_copy(oref, onesb)

    def munge(cref, b_off, base, part_rows):
        for i in range(_B // 16):
            c = cref[pl.ds(b_off + i * 16, 16)]
            inpart = (c >= base) & (c < base + part_rows)
            idx = jnp.where(inpart, c - base,
                            part_rows + (c & (_DUMP - 1)))
            idxbuf[0, pl.ds(i * 16, 16)] = idx

    def scat(ridref, base, part_rows, cref, coff):
        munge(cref, coff, base, part_rows)
        pltpu.async_copy(xcat.at[ridref], rowsb.at[0], xsem).wait()
        pltpu.sync_copy(rowsb.at[0], table.at[idxbuf.at[0]], add=True)

    def pass_body(e_base, nblk, jpart, npasses, part_rows, out_base, gather):
        base = (core * npasses + jpart) * part_rows
        q = nblk // _NTILE
        r = nblk - q * _NTILE

        nzb = (part_rows + _DUMP) // _ZB + 2
        nzi = nzb // _NTILE + jnp.where(sub < nzb - (nzb // _NTILE) * _NTILE,
                                        1, 0)

        def zbody(i, c2):
            z = sub + i * _NTILE
            pltpu.sync_copy(zbuf.at[pl.ds(0, _ZB)],
                            table.at[pl.ds(z * _ZB, _ZB)])
            return c2
        lax.fori_loop(0, nzi, zbody, 0)
        plsc.subcore_barrier()

        start_blk = sub * q + jnp.minimum(sub, r)
        start_e = e_base + start_blk * _B

        nfull = q // _CH

        def chunk_body(ch, c2):
            b0e = start_e + ch * (_CH * _B)
            pltpu.sync_copy(cols.at[pl.ds(b0e, _CH * _B)], colflat)
            if gather:
                pltpu.sync_copy(rows.at[pl.ds(b0e, _CH * _B)], rowflat)
                pltpu.async_copy(xcat.at[rowflat.at[pl.ds(0, _B)]],
                                 rowsb.at[0], gsem.at[0])
                for i in range(_CH):
                    sl = i % 2
                    if i + 1 < _CH:
                        pltpu.async_copy(
                            xcat.at[rowflat.at[pl.ds((i + 1) * _B, _B)]],
                            rowsb.at[1 - sl], gsem.at[1 - sl])
                    munge(colflat, i * _B, base, part_rows)
                    pltpu.make_async_copy(
                        xcat.at[rowflat.at[pl.ds(i * _B, _B)]],
                        rowsb.at[sl], gsem.at[sl]).wait()
                    pltpu.sync_copy(rowsb.at[sl], table.at[idxbuf.at[0]],
                                    add=True)
            else:
                for i in range(_CH):
                    munge(colflat, i * _B, base, part_rows)
                    pltpu.sync_copy(onesb, table.at[idxbuf.at[0]],
                                    add=True)
            return c2
        lax.fori_loop(0, nfull, chunk_body, 0)

        def tail_body(tb, c2):
            tbe = start_e + (nfull * _CH + tb) * _B
            pltpu.sync_copy(cols.at[pl.ds(tbe, _B)], xcol)
            if gather:
                pltpu.sync_copy(rows.at[pl.ds(tbe, _B)], xrow)
                scat(xrow, base, part_rows, xcol, 0)
            else:
                munge(xcol, 0, base, part_rows)
                pltpu.sync_copy(onesb, table.at[idxbuf.at[0]], add=True)
            return c2
        lax.fori_loop(0, q - nfull * _CH, tail_body, 0)

        @pl.when(sub < r)
        def _():
            xbe = e_base + (start_blk + q) * _B
            pltpu.sync_copy(cols.at[pl.ds(xbe, _B)], xcol)
            if gather:
                pltpu.sync_copy(rows.at[pl.ds(xbe, _B)], xrow)
                scat(xrow, base, part_rows, xcol, 0)
            else:
                munge(xcol, 0, base, part_rows)
                pltpu.sync_copy(onesb, table.at[idxbuf.at[0]], add=True)

        plsc.subcore_barrier()

        nwb = part_rows // _ZB
        nwi = nwb // _NTILE + jnp.where(sub < nwb - (nwb // _NTILE) * _NTILE,
                                        1, 0)

        def wbody(i, c2):
            z = sub + i * _NTILE
            pltpu.sync_copy(
                table.at[pl.ds(z * _ZB, _ZB)],
                out_ref.at[pl.ds(out_base + base + z * _ZB, _ZB)])
            return c2
        lax.fori_loop(0, nwi, wbody, 0)
        plsc.subcore_barrier()

    for p in passes:
        pass_body(*p)


def _make_segsum(d, with_cnt):
    passes, part_max, out_rows, e_tot, _, _ = _build_passes(d, with_cnt)
    out_type = jax.ShapeDtypeStruct((out_rows, d), jnp.float32)

    scratch = [
        pltpu.VMEM_SHARED((part_max + _DUMP + 2 * _ZB, d), jnp.float32),
        pltpu.VMEM((_CH * _B,), jnp.int32),
        pltpu.VMEM((_CH * _B,), jnp.int32),
        pltpu.VMEM((1, _B), jnp.int32),
        pltpu.VMEM((2, _B, d), jnp.float32),
        pltpu.VMEM((_B,), jnp.int32),
        pltpu.VMEM((_B,), jnp.int32),
        pltpu.VMEM((_ZB, d), jnp.float32),
        pltpu.VMEM((_B, d) if with_cnt else (8, d), jnp.float32),
        pltpu.SemaphoreType.DMA((2,)),
        pltpu.SemaphoreType.DMA,
    ]
    body = functools.partial(_segsum_body, passes, d, with_cnt)
    return pl.kernel(
        body, out_type=out_type,
        mesh=plsc.VectorSubcoreMesh(core_axis_name="c", subcore_axis_name="s"),
        scratch_types=scratch,
        compiler_params=pltpu.CompilerParams(use_tc_tiling_on_sc=False))


def _segsums(xd, eis, d, with_cnt):
    passes, _, _, e_tot, out_off, sum_rows = _build_passes(d, with_cnt)
    fn = _make_segsum(d, with_cnt)

    n_p, n_s = _NN['p'], _NN['s']
    src_off = {'p': 0, 's': n_p, 'g': n_p + n_s}
    xcat = jnp.concatenate([xd['p'], xd['s'], xd['g']], axis=0)

    rows_cat = jnp.concatenate(
        [eis[k][0] + src_off[_ETYPES[k][0]] for k in _TKEYS])
    cols_cat = jnp.concatenate([eis[k][1] for k in _TKEYS])

    zeros = jnp.zeros((_ZB, d), jnp.float32)
    args = [xcat, rows_cat, cols_cat, zeros]
    if with_cnt:
        args.append(jnp.ones((_B, d), jnp.float32))

    big = fn(*args)
    cnts = big[sum_rows:, :1] if with_cnt else None
    return big, cnts


_WCH = 2000


def _win_body(es, npads, *refs):
    cols_r = refs[:7]
    out_r = refs[7:14]
    wtab, cbuf = refs[14:]

    core = lax.axis_index("c")
    sub = lax.axis_index("s")
    wid = sub * 2 + core

    for t in range(7):
        e, npad = es[t], npads[t]

        @pl.when(wid == t)
        def _(col_ref=cols_r[t], o_ref=out_r[t], e=e, npad=npad):
            def ib(i, c):
                wtab[pl.ds(i * 16, 16)] = jnp.full((16,), -1, jnp.int32)
                return c
            lax.fori_loop(0, npad // 16, ib, 0)

            def cb(jc, c):
                pltpu.sync_copy(col_ref.at[pl.ds(jc * _WCH, _WCH)], cbuf)

                def inner(i, c2):
                    cvec = cbuf[pl.ds(i * 16, 16)]
                    evec = (lax.iota(jnp.int32, 16) + jc * _WCH + i * 16)
                    plsc.store_scatter(wtab, [cvec], evec)
                    return c2
                lax.fori_loop(0, _WCH // 16, inner, 0)
                return c
            lax.fori_loop(0, e // _WCH, cb, 0)

            def wb(i, c):
                pltpu.sync_copy(wtab.at[pl.ds(i * 512, 512)],
                                o_ref.at[pl.ds(i * 512, 512)])
                return c
            lax.fori_loop(0, npad // 512, wb, 0)


def _sc_win(cols_list, npads):
    es = [c.shape[0] for c in cols_list]
    out_type = [jax.ShapeDtypeStruct((n,), jnp.int32) for n in npads]
    scratch = [
        pltpu.VMEM((max(npads),), jnp.int32),
        pltpu.VMEM((_WCH,), jnp.int32),
    ]
    fn = pl.kernel(
        functools.partial(_win_body, es, npads),
        out_type=out_type,
        mesh=plsc.VectorSubcoreMesh(core_axis_name="c", subcore_axis_name="s"),
        scratch_types=scratch,
        compiler_params=pltpu.CompilerParams(use_tc_tiling_on_sc=False,
                                             needs_layout_passes=False))
    return fn(*cols_list)


def _gather_body(nblks, *refs):
    eas_r = refs[:7]
    win_r = refs[7:14]
    out_r = refs[14:21]
    ibuf, rowsb, gsem = refs[21:]

    core = lax.axis_index("c")
    sub = lax.axis_index("s")
    wid = sub * 2 + core

    for t in range(7):
        ea_ref, win_ref, o_ref = eas_r[t], win_r[t], out_r[t]
        nblk = nblks[t]
        q, r = nblk // 32, nblk % 32
        start = wid * q + jnp.minimum(wid, r)
        nmine = q + jnp.where(wid < r, 1, 0)
        nch = (nmine + _CH - 1) // _CH

        def chunk_body(ch, c2, ea_ref=ea_ref, win_ref=win_ref, o_ref=o_ref,
                       start=start, nmine=nmine):
            b0 = ch * _CH
            pltpu.sync_copy(
                win_ref.at[pl.ds((start + b0) * _B, _CH * _B)], ibuf)
            pltpu.async_copy(ea_ref.at[ibuf.at[pl.ds(0, _B)]],
                             rowsb.at[0], gsem.at[0])
            for i in range(_CH):
                sl = i % 2
                if i + 1 < _CH:
                    @pl.when(b0 + i + 1 < nmine)
                    def _():
                        pltpu.async_copy(
                            ea_ref.at[ibuf.at[pl.ds((i + 1) * _B, _B)]],
                            rowsb.at[1 - sl], gsem.at[1 - sl])

                @pl.when(b0 + i < nmine)
                def _():
                    pltpu.make_async_copy(
                        ea_ref.at[ibuf.at[pl.ds(i * _B, _B)]],
                        rowsb.at[sl], gsem.at[sl]).wait()
                    pltpu.sync_copy(
                        rowsb.at[sl],
                        o_ref.at[pl.ds((start + b0 + i) * _B, _B)])
            return c2
        lax.fori_loop(0, nch, chunk_body, 0)


def _sc_gather(ead_list, win_list):
    npads = [w.shape[0] for w in win_list]
    nblks = [n // _B for n in npads]
    out_type = [jax.ShapeDtypeStruct((n, 16), jnp.float32) for n in npads]
    scratch = [
        pltpu.VMEM((_CH * _B,), jnp.int32),
        pltpu.VMEM((2, _B, 16), jnp.float32),
        pltpu.SemaphoreType.DMA((2,)),
    ]
    fn = pl.kernel(
        functools.partial(_gather_body, nblks),
        out_type=out_type,
        mesh=plsc.VectorSubcoreMesh(core_axis_name="c", subcore_axis_name="s"),
        scratch_types=scratch,
        compiler_params=pltpu.CompilerParams(use_tc_tiling_on_sc=False))
    return fn(*ead_list, *win_list)


_BLK = 2000


def _row_spec(w, off=0):
    ob = off // _BLK
    return pl.BlockSpec((_BLK, w), lambda i, ob=ob: (ob + i, 0))


def _full_spec(shape):
    nd = len(shape)
    return pl.BlockSpec(shape, lambda i: (0,) * nd)


def _acc_stats(i, st_ref, y):
    ps = jnp.sum(y, 0, keepdims=True)
    pq = jnp.sum(y * y, 0, keepdims=True)
    st = jnp.concatenate([ps, pq, jnp.zeros((6, y.shape[1]), y.dtype)], 0)

    @pl.when(i == 0)
    def _():
        st_ref[...] = jnp.zeros_like(st_ref)
    st_ref[...] += st


def _mm_stats_kernel(x_ref, w_ref, b_ref, o_ref, st_ref):
    i = pl.program_id(0)
    y = jnp.dot(x_ref[...], w_ref[...],
                preferred_element_type=jnp.float32) + b_ref[...]
    o_ref[...] = y
    _acc_stats(i, st_ref, y)


def _mm_stats(x, w, b):
    n, din = x.shape
    dout = w.shape[1]
    return pl.pallas_call(
        _mm_stats_kernel,
        grid=(n // _BLK,),
        in_specs=[_row_spec(din), _full_spec((din, dout)),
                  _full_spec((1, dout))],
        out_specs=[_row_spec(dout), _full_spec((8, dout))],
        out_shape=[jax.ShapeDtypeStruct((n, dout), jnp.float32),
                   jax.ShapeDtypeStruct((8, dout), jnp.float32)],
    )(x, w, b.reshape(1, -1))


def _mm_statsonly_kernel(x_ref, w_ref, b_ref, st_ref):
    i = pl.program_id(0)
    y = jnp.dot(x_ref[...], w_ref[...],
                preferred_element_type=jnp.float32) + b_ref[...]
    _acc_stats(i, st_ref, y)


def _mm_statsonly(x, w, b):
    n, din = x.shape
    dout = w.shape[1]
    return pl.pallas_call(
        _mm_statsonly_kernel,
        grid=(n // _BLK,),
        in_specs=[_row_spec(din), _full_spec((din, dout)),
                  _full_spec((1, dout))],
        out_specs=_full_spec((8, dout)),
        out_shape=jax.ShapeDtypeStruct((8, dout), jnp.float32),
    )(x, w, b.reshape(1, -1))


def _affine(st, n, g, b, eps=1e-5, double=False):
    mu = st[0] / n
    var = st[1] / n - mu * mu
    sc = g / jnp.sqrt(var + eps)
    sh = b - mu * sc
    if double:
        sc, sh = 2 * sc, 2 * sh
    return sc, sh


def _affine_relu_kernel(y_ref, c_ref, o_ref):
    o_ref[...] = jnp.maximum(y_ref[...] * c_ref[0:1] + c_ref[1:2], 0.0)


def _affine_relu(y, sc, sh):
    n, dout = y.shape
    consts = jnp.stack([sc, sh])
    return pl.pallas_call(
        _affine_relu_kernel,
        grid=(n // _BLK,),
        in_specs=[_row_spec(dout), _full_spec((2, dout))],
        out_specs=_row_spec(dout),
        out_shape=jax.ShapeDtypeStruct((n, dout), jnp.float32),
    )(y, consts)


def _ea_apply_kernel(x_ref, w_ref, b_ref, c_ref, o_ref):
    y = jnp.dot(x_ref[...], w_ref[...],
                preferred_element_type=jnp.float32) + b_ref[...]
    o_ref[...] = jnp.maximum(y * c_ref[0:1] + c_ref[1:2], 0.0)


def _ea_apply(x, w, b, sc, sh):
    n, din = x.shape
    dout = w.shape[1]
    consts = jnp.stack([sc, sh])
    return pl.pallas_call(
        _ea_apply_kernel,
        grid=(n // _BLK,),
        in_specs=[_row_spec(din), _full_spec((din, dout)),
                  _full_spec((1, dout)), _full_spec((2, dout))],
        out_specs=_row_spec(dout),
        out_shape=jax.ShapeDtypeStruct((n, dout), jnp.float32),
    )(x, w, b.reshape(1, -1), consts)


def _phase1_kernel(sum_ref, cnt_ref, x_ref, ea_ref, m_ref, wl_ref, wr_ref,
                   bl_ref, we_ref, be_ref, wt_ref, bt_ref, wa_ref, ba_ref,
                   wg_ref, bg_ref, o_ref, st_ref):
    i = pl.program_id(0)
    agg = sum_ref[...] / cnt_ref[...]
    out = (jnp.dot(agg, wl_ref[...], preferred_element_type=jnp.float32)
           + bl_ref[...]
           + jnp.dot(x_ref[...], wr_ref[...],
                     preferred_element_type=jnp.float32))
    ea = ea_ref[...]
    emb = jnp.maximum(
        jnp.dot(ea, we_ref[...], preferred_element_type=jnp.float32)
        + be_ref[...], 0.0)
    t_emb = jnp.dot(emb, wt_ref[...],
                    preferred_element_type=jnp.float32) + bt_ref[...]
    t_attr = jnp.dot(ea, wa_ref[...],
                     preferred_element_type=jnp.float32) + ba_ref[...]
    t = t_emb + t_attr
    wg = wg_ref[...]
    o = t.shape[1]
    g0 = (jnp.dot(t, wg[o:2 * o], preferred_element_type=jnp.float32)
          + jnp.dot(t_attr, wg[2 * o:], preferred_element_type=jnp.float32)
          + bg_ref[...])
    gate = jax.nn.sigmoid(
        jnp.dot(out, wg[:o], preferred_element_type=jnp.float32) + g0)
    out2 = out + m_ref[...] * (gate * t)
    o_ref[...] = out2
    _acc_stats(i, st_ref, out2)


def _phase1(big, cnt_cat, off, x_dst, ea_w, mask, p, d):
    n = x_dst.shape[0]
    o = p['sage']['wl'].shape[1]
    return pl.pallas_call(
        _phase1_kernel,
        grid=(n // _BLK,),
        in_specs=[
            _row_spec(d, off), _row_spec(1, off), _row_spec(d),
            _row_spec(32), _row_spec(1),
            _full_spec((d, o)), _full_spec((d, o)), _full_spec((1, o)),
            _full_spec((32, 16)), _full_spec((1, 16)),
            _full_spec((16, o)), _full_spec((1, o)),
            _full_spec((32, o)), _full_spec((1, o)),
            _full_spec((3 * o, o)), _full_spec((1, o)),
        ],
        out_specs=[_row_spec(o), _full_spec((8, o))],
        out_shape=[jax.ShapeDtypeStruct((n, o), jnp.float32),
                   jax.ShapeDtypeStruct((8, o), jnp.float32)],
    )(big, cnt_cat, x_dst, ea_w, mask,
      p['sage']['wl'], p['sage']['wr'], p['sage']['bl'].reshape(1, -1),
      p['emb']['w'], p['emb']['b'].reshape(1, -1),
      p['temb']['w'], p['temb']['b'].reshape(1, -1),
      p['tattr']['w'], p['tattr']['b'].reshape(1, -1),
      p['gate']['w'], p['gate']['b'].reshape(1, -1))


def _phase2_3_kernel(a_ref, b2_ref, c2_ref, x_ref, w_ref, bl_ref, cs_ref,
                     o_ref):
    acc = (jnp.dot(x_ref[...], w_ref[...],
                   preferred_element_type=jnp.float32) + bl_ref[...])
    cs = cs_ref[...]
    for j, r in enumerate((a_ref, b2_ref, c2_ref)):
        acc += jnp.maximum(r[...] * cs[2 * j:2 * j + 1]
                           + cs[2 * j + 1:2 * j + 2], 0.0)
    o_ref[...] = jnp.maximum(acc, 0.0)


def _phase2_2_kernel(a_ref, b2_ref, x_ref, w_ref, bl_ref, cs_ref, o_ref):
    acc = (jnp.dot(x_ref[...], w_ref[...],
                   preferred_element_type=jnp.float32) + bl_ref[...])
    cs = cs_ref[...]
    for j, r in enumerate((a_ref, b2_ref)):
        acc += jnp.maximum(r[...] * cs[2 * j:2 * j + 1]
                           + cs[2 * j + 1:2 * j + 2], 0.0)
    o_ref[...] = jnp.maximum(acc, 0.0)


def _phase2(contribs, consts, x_self, w_self, bl_self, o):
    n = x_self.shape[0]
    d = x_self.shape[1]
    cs = jnp.concatenate([jnp.stack([sc, sh]) for sc, sh in consts])
    kfn = _phase2_3_kernel if len(contribs) == 3 else _phase2_2_kernel
    return pl.pallas_call(
        kfn,
        grid=(n // _BLK,),
        in_specs=[_row_spec(o) for _ in contribs]
        + [_row_spec(d), _full_spec((d, o)), _full_spec((1, o)),
           _full_spec((2 * len(contribs), o))],
        out_specs=_row_spec(o),
        out_shape=jax.ShapeDtypeStruct((n, o), jnp.float32),
    )(*contribs, x_self, w_self, bl_self.reshape(1, -1), cs)


def _hetero_pallas(xd, big, cnt_cat, meta, pl_, d, out_off):
    outs2 = {}
    stats = {}
    for k, (s, t_dst, _) in _ETYPES.items():
        win_mask, ea_w = meta[k]
        o2, st = _phase1(big, cnt_cat, out_off[k], xd[t_dst], ea_w,
                         win_mask, pl_[k], d)
        outs2[k] = o2
        stats[k] = st
    res = {}
    for t in ['p', 's', 'g']:
        ks = [k for k, v in _ETYPES.items() if v[1] == t]
        consts = []
        for k in ks:
            n_k = _NN[t]
            bn = pl_[k]['bn']
            consts.append(_affine(stats[k], n_k, bn['g'], bn['b'],
                                  double=True))
        sp = pl_['self_' + t]
        res[t] = _phase2([outs2[k] for k in ks], consts, xd[t],
                         sp['wl'] + sp['wr'], sp['bl'], 64)
    return res


def _final_lin_kernel(x_ref, w_ref, b_ref, a_ref, o_ref):
    y = x_ref[...] @ w_ref[...] + b_ref[0, 0]
    a = a_ref[0, 0]
    o_ref[...] = jnp.where(y >= 0, y, a * y)


def _final_lin(x, w, b, a):
    n = x.shape[0]
    blk = 2000
    return pl.pallas_call(
        _final_lin_kernel,
        grid=(n // blk,),
        in_specs=[
            pl.BlockSpec((blk, x.shape[1]), lambda i: (i, 0)),
            pl.BlockSpec((x.shape[1], 1), lambda i: (0, 0)),
            pl.BlockSpec((1, 1), lambda i: (0, 0)),
            pl.BlockSpec((1, 1), lambda i: (0, 0)),
        ],
        out_specs=pl.BlockSpec((blk, 1), lambda i: (i, 0)),
        out_shape=jax.ShapeDtypeStruct((n, 1), jnp.float32),
    )(x, w, b.reshape(1, 1), a.reshape(1, 1))


def kernel(x_pfas, x_sw, x_gw, eas, params, eis):
    xin = {'p': x_pfas, 's': x_sw, 'g': x_gw}
    nn = {t: v.shape[0] for t, v in xin.items()}

    xd = {}
    for t, x in xin.items():
        pr, pb = params['node_red'][t], params['node_bn'][t]
        y, st = _mm_stats(x, pr['w'], pr['b'])
        sc, sh = _affine(st, nn[t], pb['g'], pb['b'])
        xd[t] = _affine_relu(y, sc, sh)

    gpad = _CH * _B
    npads = [((nn[_ETYPES[k][1]] + gpad - 1) // gpad + 1) * gpad
             for k in _TKEYS]
    win_raw = _sc_win([eis[k][1] for k in _TKEYS], npads)
    masks, wincs = {}, []
    for i, (k, (s, d, e)) in enumerate(_ETYPES.items()):
        n = nn[d]
        masks[k] = (win_raw[i][:n] >= 0).astype(jnp.float32).reshape(-1, 1)
        wincs.append(jnp.maximum(win_raw[i], 0))

    raw = _sc_gather([eas[k] for k in _TKEYS], wincs)
    meta = {}
    for i, (k, (s, d, e)) in enumerate(_ETYPES.items()):
        pr, pb = params['edge_red'][k], params['edge_bn'][k]
        st = _mm_statsonly(eas[k], pr['w'], pr['b'])
        sc, sh = _affine(st, e, pb['g'], pb['b'])
        ea_w = _ea_apply(raw[i][:nn[d]], pr['w'], pr['b'], sc, sh)
        meta[k] = (masks[k], ea_w)

    _, _, _, _, out_off, sum_rows = _build_passes(32, True)

    big1, cnts1 = _segsums(xd, eis, 32, True)
    cnt_cat = jnp.maximum(cnts1, 1.0)
    xd = _hetero_pallas(xd, big1, cnt_cat, meta, params['conv1'], 32,
                        out_off)

    big2, _ = _segsums(xd, eis, 64, False)
    xd = _hetero_pallas(xd, big2, cnt_cat, meta, params['conv2'], 64,
                        out_off)

    w, b, a = params['lin']['w'], params['lin']['b'], params['prelu']
    gw = _final_lin(xd['g'], w, b, a)
    sw = _final_lin(xd['s'], w, b, a)
    return gw, sw, xd['p']
